# baseline (speedup 1.0000x reference)
import functools

import jax
import jax.numpy as jnp
from jax import lax
from jax.experimental import pallas as pl
from jax.experimental.pallas import tpu as pltpu
from jax.experimental.pallas import tpu_sc as plsc

B = 1024
L = 20
LP = 24
TD = 26 * 64
LANES = 16

NC = 2
NS = 16
NW = NC * NS
BW = B // NW
NCHUNK = TD // LANES


def _pool(rows_v, orow_v):
    def chunk_body(c, carry):
        for u in range(2):
            col = pl.ds((2 * c + u) * LANES, LANES)
            vals = [rows_v[r, col] for r in range(L)]
            while len(vals) > 1:
                nxt = [vals[i] + vals[i + 1] for i in range(0, len(vals) - 1, 2)]
                if len(vals) % 2:
                    nxt.append(vals[-1])
                vals = nxt
            orow_v[col] = vals[0]
        return carry

    lax.fori_loop(0, NCHUNK // 2, chunk_body, 0)


@functools.lru_cache(maxsize=1)
def _build():
    mesh = plsc.VectorSubcoreMesh(core_axis_name="c", subcore_axis_name="s")

    @functools.partial(
        pl.kernel,
        mesh=mesh,
        out_type=jax.ShapeDtypeStruct((B, TD), jnp.float32),
        scratch_types=[
            pltpu.VMEM((BW, LP), jnp.int32),
            pltpu.VMEM((LP, TD), jnp.float32),
            pltpu.VMEM((LP, TD), jnp.float32),
            pltpu.VMEM((TD,), jnp.float32),
            pltpu.VMEM((TD,), jnp.float32),
            pltpu.SemaphoreType.DMA,
            pltpu.SemaphoreType.DMA,
            pltpu.SemaphoreType.DMA,
            pltpu.SemaphoreType.DMA,
        ],
    )
    def emb_bag(tbl_hbm, idx_hbm, out_hbm, idx_v, rows0, rows1, orow0, orow1,
                gsem0, gsem1, osem0, osem1):
        wid = lax.axis_index("s") * NC + lax.axis_index("c")
        base = wid * BW
        pltpu.sync_copy(idx_hbm.at[pl.ds(base, BW)], idx_v)

        rows = (rows0, rows1)
        orow = (orow0, orow1)
        gsem = (gsem0, gsem1)
        osem = (osem0, osem1)

        def gather(b, buf):
            return pltpu.async_copy(
                tbl_hbm.at[idx_v.at[b]], rows[buf], gsem[buf])

        gather(0, 0)

        def pair_body(p, carry):
            for ph in range(2):
                b = 2 * p + ph

                @pl.when(b + 1 < BW)
                def _():
                    gather(b + 1, 1 - ph)

                pltpu.make_async_copy(tbl_hbm.at[idx_v.at[b]],
                                      rows[ph], gsem[ph]).wait()

                @pl.when(p > 0)
                def _():
                    pltpu.make_async_copy(orow[ph], out_hbm.at[base],
                                          osem[ph]).wait()

                _pool(rows[ph], orow[ph])
                pltpu.async_copy(orow[ph], out_hbm.at[base + b], osem[ph])
            return carry

        lax.fori_loop(0, BW // 2, pair_body, 0)
        pltpu.make_async_copy(orow0, out_hbm.at[base], osem0).wait()
        pltpu.make_async_copy(orow1, out_hbm.at[base], osem1).wait()

    return emb_bag


def kernel(weights, sharded_sparse_features, sharded_offsets):
    del sharded_offsets
    E = weights.shape[0]
    tbl = weights.reshape(E, TD)
    idx = sharded_sparse_features.reshape(B, L)
    idx_pad = jnp.pad(idx, ((0, 0), (0, LP - L)))
    out = _build()(tbl, idx_pad)
    return out.reshape(B, 26, 64)
